# baseline (speedup 1.0000x reference)
import jax
import jax.numpy as jnp
from jax.experimental import pallas as pl
from jax.experimental.pallas import tpu as pltpu


def _dropblock_body(mask_ref, x_ref, o_ref, *, bs, H, W):
    m = mask_ref[:]
    pw = jnp.pad(m, ((0, 0), (bs - 1, 0)))
    r = m
    for j in range(1, bs):
        r = jnp.maximum(r, pw[:, bs - 1 - j : bs - 1 - j + W])
    ph = jnp.pad(r, ((bs - 1, 0), (0, 0)))
    b = r
    for i in range(1, bs):
        b = jnp.maximum(b, ph[bs - 1 - i : bs - 1 - i + H, :])
    bm = 1.0 - b
    scale = jnp.float32(H * W) / jnp.sum(bm)
    o_ref[:] = x_ref[:] * (bm * scale)[None, None, :, :]


def kernel(x, block_size, feat_size, drop_rate):
    B, C, H, W = x.shape
    bs = 7
    gamma = drop_rate / (block_size ** 2) * (
        (feat_size ** 2) / ((feat_size - block_size + 1) ** 2)
    )
    mkey = jax.random.fold_in(jax.random.key(0), 1)
    mask = jax.random.bernoulli(mkey, gamma, (H, W)).astype(jnp.float32)

    K = 128
    out = pl.pallas_call(
        lambda mask_ref, x_ref, o_ref: _dropblock_body(
            mask_ref, x_ref, o_ref, bs=bs, H=H, W=W
        ),
        grid=(B, C // K),
        in_specs=[
            pl.BlockSpec((H, W), lambda b, c: (0, 0)),
            pl.BlockSpec((1, K, H, W), lambda b, c: (b, c, 0, 0)),
        ],
        out_specs=pl.BlockSpec((1, K, H, W), lambda b, c: (b, c, 0, 0)),
        out_shape=jax.ShapeDtypeStruct((B, C, H, W), x.dtype),
        compiler_params=pltpu.CompilerParams(
            dimension_semantics=("parallel", "parallel"),
        ),
    )(mask, x)
    return out


# speedup vs baseline: 4.5521x; 4.5521x over previous
import jax
import jax.numpy as jnp
from jax import lax
from jax.experimental import pallas as pl
from jax.experimental.pallas import tpu as pltpu


def _dropblock_body(mask_ref, x_ref, o_ref, m_ref, *, bs, H, W):
    HW = H * W

    @pl.when(pl.program_id(0) == 0)
    def _():
        m = mask_ref[:]
        wcol = lax.broadcasted_iota(jnp.int32, (HW, 1), 0) & (W - 1)
        r = m
        for j in range(1, bs):
            sh = jnp.pad(m, ((j, 0), (0, 0)))[:HW, :]
            r = jnp.maximum(r, jnp.where(wcol >= j, sh, 0.0))
        b = r
        for i in range(1, bs):
            sh = jnp.pad(r, ((W * i, 0), (0, 0)))[:HW, :]
            b = jnp.maximum(b, sh)
        bm = 1.0 - b
        scale = jnp.float32(HW) / jnp.sum(bm)
        m_ref[:] = bm * scale

    o_ref[:] = x_ref[:] * m_ref[:]


def kernel(x, block_size, feat_size, drop_rate):
    B, C, H, W = x.shape
    bs = 7
    gamma = drop_rate / (block_size ** 2) * (
        (feat_size ** 2) / ((feat_size - block_size + 1) ** 2)
    )
    mkey = jax.random.fold_in(jax.random.key(0), 1)
    mask = jax.random.bernoulli(mkey, gamma, (H, W)).astype(jnp.float32)

    HW = H * W
    xt = x.transpose(0, 2, 3, 1).reshape(B * HW, C)

    out = pl.pallas_call(
        lambda mask_ref, x_ref, o_ref, m_ref: _dropblock_body(
            mask_ref, x_ref, o_ref, m_ref, bs=bs, H=H, W=W
        ),
        grid=(B,),
        in_specs=[
            pl.BlockSpec((HW, 1), lambda i: (0, 0)),
            pl.BlockSpec((HW, C), lambda i: (i, 0)),
        ],
        out_specs=pl.BlockSpec((HW, C), lambda i: (i, 0)),
        out_shape=jax.ShapeDtypeStruct((B * HW, C), x.dtype),
        scratch_shapes=[pltpu.VMEM((HW, 1), jnp.float32)],
        compiler_params=pltpu.CompilerParams(
            dimension_semantics=("arbitrary",),
        ),
    )(mask.reshape(HW, 1), xt)
    return out.reshape(B, H, W, C).transpose(0, 3, 1, 2)
